# baseline (speedup 1.0000x reference)
import functools

import jax
import jax.numpy as jnp
from jax import lax
from jax.experimental import pallas as pl
from jax.experimental.pallas import tpu as pltpu
from jax.experimental.pallas import tpu_sc as plsc

VOCAB = 1000000
EMB = 32
BATCH = 4096
SEQ = 200

_NC, _NS = 2, 16
_NW = _NC * _NS
_BW = BATCH // _NW
_NSUPER = VOCAB * EMB // 128

_TCOLS = 512
_NFULL = VOCAB // _TCOLS
_TAIL = VOCAB - _NFULL * _TCOLS
_KMAX = -(-_NFULL // _NW)


def _tr_body(tt_hbm, tail_hbm, scr_hbm, src0, src1, dstv, tailsrc, rsem0, rsem1):
    wid = lax.axis_index("s") * _NC + lax.axis_index("c")
    src = (src0, src1)
    rsem = (rsem0, rsem1)
    iota16 = lax.broadcasted_iota(jnp.int32, (16,), 0)

    def read_blk(bid, b):
        return pltpu.make_async_copy(
            tt_hbm.at[:, pl.ds(bid * _TCOLS, _TCOLS)], src[b], rsem[b])

    read_blk(wid, 0).start()

    def transpose_block(sbuf, ncols):
        @plsc.parallel_loop(0, ncols // 16, unroll=2)
        def _tr(g):
            v0 = g * 16
            rowv = (v0 + iota16) >> 2
            colb = ((v0 + iota16) & 3) << 5

            @plsc.parallel_loop(0, EMB, unroll=8)
            def _tr_e(e):
                vals = sbuf[e, pl.ds(v0, 16)]
                plsc.store_scatter(dstv, [rowv, colb + e], vals)

    def step(k, carry):
        bid = wid + k * _NW

        @pl.when(bid < _NFULL)
        def _():
            b = lax.rem(k, 2)

            @pl.when(b == 0)
            def _():
                _step_buf(k, bid, 0)

            @pl.when(b == 1)
            def _():
                _step_buf(k, bid, 1)

        return carry

    def _step_buf(k, bid, b):
        read_blk(bid, b).wait()
        bid_next = bid + _NW

        @pl.when(bid_next < _NFULL)
        def _():
            read_blk(bid_next, 1 - b).start()

        transpose_block(src[b], _TCOLS)
        pltpu.sync_copy(dstv, scr_hbm.at[pl.ds(bid * (_TCOLS // 4), _TCOLS // 4)])

    lax.fori_loop(0, _KMAX, step, 0)

    @pl.when(wid == _NW - 1)
    def _():
        pltpu.sync_copy(tail_hbm, tailsrc)
        pltpu.sync_copy(
            tailsrc, scr_hbm.at[pl.ds(_NFULL * (_TCOLS // 4), _TAIL // 4)])


def _emb_body(xt_hbm, tw_hbm, out_hbm, xblk, sup, grows0, grows1,
              outb0, outb1, gsem0, gsem1, osem0, osem1):
    wid = lax.axis_index("s") * _NC + lax.axis_index("c")
    b0 = wid * _BW
    grows = (grows0, grows1)
    outb = (outb0, outb1)
    gsem = (gsem0, gsem1)
    osem = (osem0, osem1)

    pltpu.sync_copy(xt_hbm.at[:, pl.ds(b0, _BW)], xblk)

    @plsc.parallel_loop(0, SEQ)
    def _sup_body(s):
        for c in range(_BW // 16):
            xv = xblk[s, pl.ds(16 * c, 16)]
            sup[s, pl.ds(16 * c, 16)] = xv >> 2
            xblk[s, pl.ds(16 * c, 16)] = (xv & 3) << 5

    def gather_s(s, buf):
        return pltpu.make_async_copy(
            tw_hbm.at[sup.at[s]], grows[buf], gsem[buf])

    def store_s(s, buf):
        return pltpu.make_async_copy(
            outb[buf], out_hbm.at[s, :, pl.ds(b0, _BW)], osem[buf])

    gather_s(0, 0).start()
    iota16 = lax.broadcasted_iota(jnp.int32, (16,), 0)

    def _step_buf(s, b):
        gather_s(s, b).wait()

        @pl.when(s + 1 < SEQ)
        def _():
            gather_s(s + 1, 1 - b).start()

        @pl.when(s >= 2)
        def _():
            store_s(s, b).wait()

        for c in range(_BW // 16):
            jv = iota16 + (16 * c)
            cb = xblk[s, pl.ds(16 * c, 16)]

            @plsc.parallel_loop(0, EMB, unroll=8)
            def _ext(e):
                vals = plsc.load_gather(grows[b], [jv, cb + e])
                outb[b][e, pl.ds(16 * c, 16)] = vals

        store_s(s, b).start()

    def step(s, carry):
        @pl.when(lax.rem(s, 2) == 0)
        def _():
            _step_buf(s, 0)

        @pl.when(lax.rem(s, 2) == 1)
        def _():
            _step_buf(s, 1)

        return carry

    lax.fori_loop(0, SEQ, step, 0)

    store_s(SEQ - 2, 0).wait()
    store_s(SEQ - 1, 1).wait()


@functools.cache
def _build_tr():
    return pl.kernel(
        _tr_body,
        mesh=plsc.VectorSubcoreMesh(core_axis_name="c", subcore_axis_name="s"),
        out_type=jax.ShapeDtypeStruct((_NSUPER, 128), jnp.float32),
        scratch_types=[
            pltpu.VMEM((EMB, _TCOLS), jnp.float32),
            pltpu.VMEM((EMB, _TCOLS), jnp.float32),
            pltpu.VMEM((_TCOLS // 4, 128), jnp.float32),
            pltpu.VMEM((_TAIL // 4, 128), jnp.float32),
            pltpu.SemaphoreType.DMA,
            pltpu.SemaphoreType.DMA,
        ],
        compiler_params=pltpu.CompilerParams(
            use_tc_tiling_on_sc=True, needs_layout_passes=False),
    )


@functools.cache
def _build_emb():
    return pl.kernel(
        _emb_body,
        mesh=plsc.VectorSubcoreMesh(core_axis_name="c", subcore_axis_name="s"),
        out_type=jax.ShapeDtypeStruct((SEQ, EMB, BATCH), jnp.float32),
        scratch_types=[
            pltpu.VMEM((SEQ, _BW), jnp.int32),
            pltpu.VMEM((SEQ, _BW), jnp.int32),
            pltpu.VMEM((_BW, 128), jnp.float32),
            pltpu.VMEM((_BW, 128), jnp.float32),
            pltpu.VMEM((EMB, _BW), jnp.float32),
            pltpu.VMEM((EMB, _BW), jnp.float32),
            pltpu.SemaphoreType.DMA,
            pltpu.SemaphoreType.DMA,
            pltpu.SemaphoreType.DMA,
            pltpu.SemaphoreType.DMA,
        ],
        compiler_params=pltpu.CompilerParams(
            use_tc_tiling_on_sc=True, needs_layout_passes=False),
    )


def kernel(x, table):
    xt = jnp.transpose(x)
    tt = jnp.transpose(table)
    tail = table[_NFULL * _TCOLS:].reshape(_TAIL // 4, 128)
    tw = _build_tr()(tt, tail)
    o = _build_emb()(xt, tw)
    return jnp.transpose(o, (2, 0, 1))


# speedup vs baseline: 1.0889x; 1.0889x over previous
import functools

import jax
import jax.numpy as jnp
from jax import lax
from jax.experimental import pallas as pl
from jax.experimental.pallas import tpu as pltpu
from jax.experimental.pallas import tpu_sc as plsc

VOCAB = 1000000
EMB = 32
BATCH = 4096
SEQ = 200

_NC, _NS = 2, 16
_NW = _NC * _NS
_BW = BATCH // _NW
_NSUPER = VOCAB * EMB // 128

_TCOLS = 512
_NFULL = VOCAB // _TCOLS
_TAIL = VOCAB - _NFULL * _TCOLS
_KMAX = -(-_NFULL // _NW)


def _tr_body(tt_hbm, tail_hbm, scr_hbm, src0, src1, dstv0, dstv1, tailsrc,
             rsem0, rsem1, wsem0, wsem1):
    wid = lax.axis_index("s") * _NC + lax.axis_index("c")
    src = (src0, src1)
    dstv = (dstv0, dstv1)
    rsem = (rsem0, rsem1)
    wsem = (wsem0, wsem1)
    iota16 = lax.broadcasted_iota(jnp.int32, (16,), 0)

    def read_blk(bid, b):
        return pltpu.make_async_copy(
            tt_hbm.at[:, pl.ds(bid * _TCOLS, _TCOLS)], src[b], rsem[b])

    def write_blk(bid, b):
        return pltpu.make_async_copy(
            dstv[b], scr_hbm.at[pl.ds(bid * (_TCOLS // 4), _TCOLS // 4)],
            wsem[b])

    read_blk(wid, 0).start()

    def transpose_block(sbuf, dbuf, ncols):
        @plsc.parallel_loop(0, ncols, unroll=4)
        def _tr(v):
            vq = v >> 2
            cb = (v & 3) << 5
            vcol = jnp.zeros((16,), jnp.int32) + v
            for eh in range(2):
                vals = plsc.load_gather(sbuf, [iota16 + 16 * eh, vcol])
                dbuf[vq, pl.ds(cb + 16 * eh, 16)] = vals

    def step(k, carry):
        bid = wid + k * _NW

        @pl.when(bid < _NFULL)
        def _():
            b = lax.rem(k, 2)

            @pl.when(b == 0)
            def _():
                _step_buf(k, bid, 0)

            @pl.when(b == 1)
            def _():
                _step_buf(k, bid, 1)

        return carry

    def _step_buf(k, bid, b):
        read_blk(bid, b).wait()
        bid_next = bid + _NW

        @pl.when(bid_next < _NFULL)
        def _():
            read_blk(bid_next, 1 - b).start()

        @pl.when(k >= 2)
        def _():
            write_blk(bid, b).wait()

        transpose_block(src[b], dstv[b], _TCOLS)
        write_blk(bid, b).start()

    lax.fori_loop(0, _KMAX, step, 0)

    write_blk(wid, 0).wait()
    write_blk(wid, 1).wait()

    @pl.when(wid == _NW - 1)
    def _():
        pltpu.sync_copy(tail_hbm, tailsrc)
        pltpu.sync_copy(
            tailsrc, scr_hbm.at[pl.ds(_NFULL * (_TCOLS // 4), _TAIL // 4)])


def _emb_body(xt_hbm, tw_hbm, out_hbm, xblk, sup, grows0, grows1,
              outb0, outb1, gsem0, gsem1, osem0, osem1):
    wid = lax.axis_index("s") * _NC + lax.axis_index("c")
    b0 = wid * _BW
    grows = (grows0, grows1)
    outb = (outb0, outb1)
    gsem = (gsem0, gsem1)
    osem = (osem0, osem1)

    pltpu.sync_copy(xt_hbm.at[:, pl.ds(b0, _BW)], xblk)

    @plsc.parallel_loop(0, SEQ)
    def _sup_body(s):
        for c in range(_BW // 16):
            xv = xblk[s, pl.ds(16 * c, 16)]
            sup[s, pl.ds(16 * c, 16)] = xv >> 2
            xblk[s, pl.ds(16 * c, 16)] = (xv & 3) << 5

    def gather_s(s, buf):
        return pltpu.make_async_copy(
            tw_hbm.at[sup.at[s]], grows[buf], gsem[buf])

    def store_s(s, buf):
        return pltpu.make_async_copy(
            outb[buf], out_hbm.at[s, :, pl.ds(b0, _BW)], osem[buf])

    gather_s(0, 0).start()
    iota16 = lax.broadcasted_iota(jnp.int32, (16,), 0)

    def _step_buf(s, b):
        gather_s(s, b).wait()

        @pl.when(s + 1 < SEQ)
        def _():
            gather_s(s + 1, 1 - b).start()

        @pl.when(s >= 2)
        def _():
            store_s(s, b).wait()

        for c in range(_BW // 16):
            jv = iota16 + (16 * c)
            cb = xblk[s, pl.ds(16 * c, 16)]

            @plsc.parallel_loop(0, EMB, unroll=8)
            def _ext(e):
                vals = plsc.load_gather(grows[b], [jv, cb + e])
                outb[b][e, pl.ds(16 * c, 16)] = vals

        store_s(s, b).start()

    def step(s, carry):
        @pl.when(lax.rem(s, 2) == 0)
        def _():
            _step_buf(s, 0)

        @pl.when(lax.rem(s, 2) == 1)
        def _():
            _step_buf(s, 1)

        return carry

    lax.fori_loop(0, SEQ, step, 0)

    store_s(SEQ - 2, 0).wait()
    store_s(SEQ - 1, 1).wait()


@functools.cache
def _build_tr():
    return pl.kernel(
        _tr_body,
        mesh=plsc.VectorSubcoreMesh(core_axis_name="c", subcore_axis_name="s"),
        out_type=jax.ShapeDtypeStruct((_NSUPER, 128), jnp.float32),
        scratch_types=[
            pltpu.VMEM((EMB, _TCOLS), jnp.float32),
            pltpu.VMEM((EMB, _TCOLS), jnp.float32),
            pltpu.VMEM((_TCOLS // 4, 128), jnp.float32),
            pltpu.VMEM((_TCOLS // 4, 128), jnp.float32),
            pltpu.VMEM((_TAIL // 4, 128), jnp.float32),
            pltpu.SemaphoreType.DMA,
            pltpu.SemaphoreType.DMA,
            pltpu.SemaphoreType.DMA,
            pltpu.SemaphoreType.DMA,
        ],
        compiler_params=pltpu.CompilerParams(
            use_tc_tiling_on_sc=True, needs_layout_passes=False),
    )


@functools.cache
def _build_emb():
    return pl.kernel(
        _emb_body,
        mesh=plsc.VectorSubcoreMesh(core_axis_name="c", subcore_axis_name="s"),
        out_type=jax.ShapeDtypeStruct((SEQ, EMB, BATCH), jnp.float32),
        scratch_types=[
            pltpu.VMEM((SEQ, _BW), jnp.int32),
            pltpu.VMEM((SEQ, _BW), jnp.int32),
            pltpu.VMEM((_BW, 128), jnp.float32),
            pltpu.VMEM((_BW, 128), jnp.float32),
            pltpu.VMEM((EMB, _BW), jnp.float32),
            pltpu.VMEM((EMB, _BW), jnp.float32),
            pltpu.SemaphoreType.DMA,
            pltpu.SemaphoreType.DMA,
            pltpu.SemaphoreType.DMA,
            pltpu.SemaphoreType.DMA,
        ],
        compiler_params=pltpu.CompilerParams(
            use_tc_tiling_on_sc=True, needs_layout_passes=False),
    )


def kernel(x, table):
    xt = jnp.transpose(x)
    tt = jnp.transpose(table)
    tail = table[_NFULL * _TCOLS:].reshape(_TAIL // 4, 128)
    tw = _build_tr()(tt, tail)
    o = _build_emb()(xt, tw)
    return jnp.transpose(o, (2, 0, 1))


# speedup vs baseline: 1.0892x; 1.0003x over previous
import functools

import jax
import jax.numpy as jnp
from jax import lax
from jax.experimental import pallas as pl
from jax.experimental.pallas import tpu as pltpu
from jax.experimental.pallas import tpu_sc as plsc

VOCAB = 1000000
EMB = 32
BATCH = 4096
SEQ = 200

_NC, _NS = 2, 16
_NW = _NC * _NS
_BW = BATCH // _NW
_NSUPER = VOCAB * EMB // 128

_TCOLS = 512
_NFULL = VOCAB // _TCOLS
_TAIL = VOCAB - _NFULL * _TCOLS
_KMAX = -(-_NFULL // _NW)


def _tr_body(tt_hbm, tail_hbm, scr_hbm, src0, src1, dstv0, dstv1, tailsrc,
             rsem0, rsem1, wsem0, wsem1):
    wid = lax.axis_index("s") * _NC + lax.axis_index("c")
    src = (src0, src1)
    dstv = (dstv0, dstv1)
    rsem = (rsem0, rsem1)
    wsem = (wsem0, wsem1)
    iota16 = lax.broadcasted_iota(jnp.int32, (16,), 0)

    def read_blk(bid, b):
        return pltpu.make_async_copy(
            tt_hbm.at[:, pl.ds(bid * _TCOLS, _TCOLS)], src[b], rsem[b])

    def write_blk(bid, b):
        return pltpu.make_async_copy(
            dstv[b], scr_hbm.at[pl.ds(bid * (_TCOLS // 4), _TCOLS // 4)],
            wsem[b])

    read_blk(wid, 0).start()

    def transpose_block(sbuf, dbuf, ncols):
        @plsc.parallel_loop(0, ncols, unroll=8)
        def _tr(v):
            vq = v >> 2
            cb = (v & 3) << 5
            vcol = jnp.zeros((16,), jnp.int32) + v
            for eh in range(2):
                vals = plsc.load_gather(sbuf, [iota16 + 16 * eh, vcol])
                dbuf[vq, pl.ds(cb + 16 * eh, 16)] = vals

    def step(k, carry):
        bid = wid + k * _NW

        @pl.when(bid < _NFULL)
        def _():
            b = lax.rem(k, 2)

            @pl.when(b == 0)
            def _():
                _step_buf(k, bid, 0)

            @pl.when(b == 1)
            def _():
                _step_buf(k, bid, 1)

        return carry

    def _step_buf(k, bid, b):
        read_blk(bid, b).wait()
        bid_next = bid + _NW

        @pl.when(bid_next < _NFULL)
        def _():
            read_blk(bid_next, 1 - b).start()

        @pl.when(k >= 2)
        def _():
            write_blk(bid, b).wait()

        transpose_block(src[b], dstv[b], _TCOLS)
        write_blk(bid, b).start()

    lax.fori_loop(0, _KMAX, step, 0)

    write_blk(wid, 0).wait()
    write_blk(wid, 1).wait()

    @pl.when(wid == _NW - 1)
    def _():
        pltpu.sync_copy(tail_hbm, tailsrc)
        pltpu.sync_copy(
            tailsrc, scr_hbm.at[pl.ds(_NFULL * (_TCOLS // 4), _TAIL // 4)])


def _emb_body(xt_hbm, tw_hbm, out_hbm, xblk, sup, grows0, grows1, grows2,
              outb0, outb1, outb2, gsem0, gsem1, gsem2, osem0, osem1, osem2):
    wid = lax.axis_index("s") * _NC + lax.axis_index("c")
    b0 = wid * _BW
    grows = (grows0, grows1, grows2)
    outb = (outb0, outb1, outb2)
    gsem = (gsem0, gsem1, gsem2)
    osem = (osem0, osem1, osem2)

    pltpu.sync_copy(xt_hbm.at[:, pl.ds(b0, _BW)], xblk)

    @plsc.parallel_loop(0, SEQ)
    def _sup_body(s):
        for c in range(_BW // 16):
            xv = xblk[s, pl.ds(16 * c, 16)]
            sup[s, pl.ds(16 * c, 16)] = xv >> 2
            xblk[s, pl.ds(16 * c, 16)] = (xv & 3) << 5

    def gather_s(s, buf):
        return pltpu.make_async_copy(
            tw_hbm.at[sup.at[s]], grows[buf], gsem[buf])

    def store_s(s, buf):
        return pltpu.make_async_copy(
            outb[buf], out_hbm.at[s, :, pl.ds(b0, _BW)], osem[buf])

    gather_s(0, 0).start()
    gather_s(1, 1).start()
    iota16 = lax.broadcasted_iota(jnp.int32, (16,), 0)

    def _step_buf(s, b):
        gather_s(s, b).wait()

        @pl.when(s + 2 < SEQ)
        def _():
            gather_s(s + 2, (b + 2) % 3).start()

        @pl.when(s >= 3)
        def _():
            store_s(s, b).wait()

        for c in range(_BW // 16):
            jv = iota16 + (16 * c)
            cb = xblk[s, pl.ds(16 * c, 16)]

            @plsc.parallel_loop(0, EMB, unroll=8)
            def _ext(e):
                vals = plsc.load_gather(grows[b], [jv, cb + e])
                outb[b][e, pl.ds(16 * c, 16)] = vals

        store_s(s, b).start()

    def step(s, carry):
        @pl.when(lax.rem(s, 3) == 0)
        def _():
            _step_buf(s, 0)

        @pl.when(lax.rem(s, 3) == 1)
        def _():
            _step_buf(s, 1)

        @pl.when(lax.rem(s, 3) == 2)
        def _():
            _step_buf(s, 2)

        return carry

    lax.fori_loop(0, SEQ, step, 0)

    store_s(SEQ - 3, (SEQ - 3) % 3).wait()
    store_s(SEQ - 2, (SEQ - 2) % 3).wait()
    store_s(SEQ - 1, (SEQ - 1) % 3).wait()


@functools.cache
def _build_tr():
    return pl.kernel(
        _tr_body,
        mesh=plsc.VectorSubcoreMesh(core_axis_name="c", subcore_axis_name="s"),
        out_type=jax.ShapeDtypeStruct((_NSUPER, 128), jnp.float32),
        scratch_types=[
            pltpu.VMEM((EMB, _TCOLS), jnp.float32),
            pltpu.VMEM((EMB, _TCOLS), jnp.float32),
            pltpu.VMEM((_TCOLS // 4, 128), jnp.float32),
            pltpu.VMEM((_TCOLS // 4, 128), jnp.float32),
            pltpu.VMEM((_TAIL // 4, 128), jnp.float32),
            pltpu.SemaphoreType.DMA,
            pltpu.SemaphoreType.DMA,
            pltpu.SemaphoreType.DMA,
            pltpu.SemaphoreType.DMA,
        ],
        compiler_params=pltpu.CompilerParams(
            use_tc_tiling_on_sc=True, needs_layout_passes=False),
    )


@functools.cache
def _build_emb():
    return pl.kernel(
        _emb_body,
        mesh=plsc.VectorSubcoreMesh(core_axis_name="c", subcore_axis_name="s"),
        out_type=jax.ShapeDtypeStruct((SEQ, EMB, BATCH), jnp.float32),
        scratch_types=[
            pltpu.VMEM((SEQ, _BW), jnp.int32),
            pltpu.VMEM((SEQ, _BW), jnp.int32),
            pltpu.VMEM((_BW, 128), jnp.float32),
            pltpu.VMEM((_BW, 128), jnp.float32),
            pltpu.VMEM((_BW, 128), jnp.float32),
            pltpu.VMEM((EMB, _BW), jnp.float32),
            pltpu.VMEM((EMB, _BW), jnp.float32),
            pltpu.VMEM((EMB, _BW), jnp.float32),
            pltpu.SemaphoreType.DMA,
            pltpu.SemaphoreType.DMA,
            pltpu.SemaphoreType.DMA,
            pltpu.SemaphoreType.DMA,
            pltpu.SemaphoreType.DMA,
            pltpu.SemaphoreType.DMA,
        ],
        compiler_params=pltpu.CompilerParams(
            use_tc_tiling_on_sc=True, needs_layout_passes=False),
    )


def kernel(x, table):
    xt = jnp.transpose(x)
    tt = jnp.transpose(table)
    tail = table[_NFULL * _TCOLS:].reshape(_TAIL // 4, 128)
    tw = _build_tr()(tt, tail)
    o = _build_emb()(xt, tw)
    return jnp.transpose(o, (2, 0, 1))


# speedup vs baseline: 1.0940x; 1.0044x over previous
import functools

import jax
import jax.numpy as jnp
from jax import lax
from jax.experimental import pallas as pl
from jax.experimental.pallas import tpu as pltpu
from jax.experimental.pallas import tpu_sc as plsc

VOCAB = 1000000
EMB = 32
BATCH = 4096
SEQ = 200

_NC, _NS = 2, 16
_NW = _NC * _NS
_BW = BATCH // _NW
_NSUPER = VOCAB * EMB // 128

_TCOLS = 768
_NFULL = VOCAB // _TCOLS
_TAIL = VOCAB - _NFULL * _TCOLS
_KMAX = -(-_NFULL // _NW)


def _tr_body(tt_hbm, tail_hbm, scr_hbm, src0, src1, dstv0, dstv1, tailsrc,
             rsem0, rsem1, wsem0, wsem1):
    wid = lax.axis_index("s") * _NC + lax.axis_index("c")
    src = (src0, src1)
    dstv = (dstv0, dstv1)
    rsem = (rsem0, rsem1)
    wsem = (wsem0, wsem1)
    iota16 = lax.broadcasted_iota(jnp.int32, (16,), 0)

    def read_blk(bid, b):
        return pltpu.make_async_copy(
            tt_hbm.at[:, pl.ds(bid * _TCOLS, _TCOLS)], src[b], rsem[b])

    def write_blk(bid, b):
        return pltpu.make_async_copy(
            dstv[b], scr_hbm.at[pl.ds(bid * (_TCOLS // 4), _TCOLS // 4)],
            wsem[b])

    read_blk(wid, 0).start()

    def transpose_block(sbuf, dbuf, ncols):
        @plsc.parallel_loop(0, ncols, unroll=8)
        def _tr(v):
            vq = v >> 2
            cb = (v & 3) << 5
            vcol = jnp.zeros((16,), jnp.int32) + v
            for eh in range(2):
                vals = plsc.load_gather(sbuf, [iota16 + 16 * eh, vcol])
                dbuf[vq, pl.ds(cb + 16 * eh, 16)] = vals

    def step(k, carry):
        bid = wid + k * _NW

        @pl.when(bid < _NFULL)
        def _():
            b = lax.rem(k, 2)

            @pl.when(b == 0)
            def _():
                _step_buf(k, bid, 0)

            @pl.when(b == 1)
            def _():
                _step_buf(k, bid, 1)

        return carry

    def _step_buf(k, bid, b):
        read_blk(bid, b).wait()
        bid_next = bid + _NW

        @pl.when(bid_next < _NFULL)
        def _():
            read_blk(bid_next, 1 - b).start()

        @pl.when(k >= 2)
        def _():
            write_blk(bid, b).wait()

        transpose_block(src[b], dstv[b], _TCOLS)
        write_blk(bid, b).start()

    lax.fori_loop(0, _KMAX, step, 0)

    write_blk(wid, 0).wait()
    write_blk(wid, 1).wait()

    @pl.when(wid == _NW - 1)
    def _():
        pltpu.sync_copy(tail_hbm, tailsrc)
        pltpu.sync_copy(
            tailsrc, scr_hbm.at[pl.ds(_NFULL * (_TCOLS // 4), _TAIL // 4)])


def _emb_body(xt_hbm, tw_hbm, out_hbm, xblk, sup, grows0, grows1, grows2,
              outb0, outb1, outb2, gsem0, gsem1, gsem2, osem0, osem1, osem2):
    wid = lax.axis_index("s") * _NC + lax.axis_index("c")
    b0 = wid * _BW
    grows = (grows0, grows1, grows2)
    outb = (outb0, outb1, outb2)
    gsem = (gsem0, gsem1, gsem2)
    osem = (osem0, osem1, osem2)

    pltpu.sync_copy(xt_hbm.at[:, pl.ds(b0, _BW)], xblk)

    @plsc.parallel_loop(0, SEQ)
    def _sup_body(s):
        for c in range(_BW // 16):
            xv = xblk[s, pl.ds(16 * c, 16)]
            sup[s, pl.ds(16 * c, 16)] = xv >> 2
            xblk[s, pl.ds(16 * c, 16)] = (xv & 3) << 5

    def gather_s(s, buf):
        return pltpu.make_async_copy(
            tw_hbm.at[sup.at[s]], grows[buf], gsem[buf])

    def store_s(s, buf):
        return pltpu.make_async_copy(
            outb[buf], out_hbm.at[s, :, pl.ds(b0, _BW)], osem[buf])

    gather_s(0, 0).start()
    gather_s(1, 1).start()
    iota16 = lax.broadcasted_iota(jnp.int32, (16,), 0)

    def _step_buf(s, b):
        gather_s(s, b).wait()

        @pl.when(s + 2 < SEQ)
        def _():
            gather_s(s + 2, (b + 2) % 3).start()

        @pl.when(s >= 3)
        def _():
            store_s(s, b).wait()

        for c in range(_BW // 16):
            jv = iota16 + (16 * c)
            cb = xblk[s, pl.ds(16 * c, 16)]

            @plsc.parallel_loop(0, EMB, unroll=8)
            def _ext(e):
                vals = plsc.load_gather(grows[b], [jv, cb + e])
                outb[b][e, pl.ds(16 * c, 16)] = vals

        store_s(s, b).start()

    def step(s, carry):
        @pl.when(lax.rem(s, 3) == 0)
        def _():
            _step_buf(s, 0)

        @pl.when(lax.rem(s, 3) == 1)
        def _():
            _step_buf(s, 1)

        @pl.when(lax.rem(s, 3) == 2)
        def _():
            _step_buf(s, 2)

        return carry

    lax.fori_loop(0, SEQ, step, 0)

    store_s(SEQ - 3, (SEQ - 3) % 3).wait()
    store_s(SEQ - 2, (SEQ - 2) % 3).wait()
    store_s(SEQ - 1, (SEQ - 1) % 3).wait()


@functools.cache
def _build_tr():
    return pl.kernel(
        _tr_body,
        mesh=plsc.VectorSubcoreMesh(core_axis_name="c", subcore_axis_name="s"),
        out_type=jax.ShapeDtypeStruct((_NSUPER, 128), jnp.float32),
        scratch_types=[
            pltpu.VMEM((EMB, _TCOLS), jnp.float32),
            pltpu.VMEM((EMB, _TCOLS), jnp.float32),
            pltpu.VMEM((_TCOLS // 4, 128), jnp.float32),
            pltpu.VMEM((_TCOLS // 4, 128), jnp.float32),
            pltpu.VMEM((_TAIL // 4, 128), jnp.float32),
            pltpu.SemaphoreType.DMA,
            pltpu.SemaphoreType.DMA,
            pltpu.SemaphoreType.DMA,
            pltpu.SemaphoreType.DMA,
        ],
        compiler_params=pltpu.CompilerParams(
            use_tc_tiling_on_sc=True, needs_layout_passes=False),
    )


@functools.cache
def _build_emb():
    return pl.kernel(
        _emb_body,
        mesh=plsc.VectorSubcoreMesh(core_axis_name="c", subcore_axis_name="s"),
        out_type=jax.ShapeDtypeStruct((SEQ, EMB, BATCH), jnp.float32),
        scratch_types=[
            pltpu.VMEM((SEQ, _BW), jnp.int32),
            pltpu.VMEM((SEQ, _BW), jnp.int32),
            pltpu.VMEM((_BW, 128), jnp.float32),
            pltpu.VMEM((_BW, 128), jnp.float32),
            pltpu.VMEM((_BW, 128), jnp.float32),
            pltpu.VMEM((EMB, _BW), jnp.float32),
            pltpu.VMEM((EMB, _BW), jnp.float32),
            pltpu.VMEM((EMB, _BW), jnp.float32),
            pltpu.SemaphoreType.DMA,
            pltpu.SemaphoreType.DMA,
            pltpu.SemaphoreType.DMA,
            pltpu.SemaphoreType.DMA,
            pltpu.SemaphoreType.DMA,
            pltpu.SemaphoreType.DMA,
        ],
        compiler_params=pltpu.CompilerParams(
            use_tc_tiling_on_sc=True, needs_layout_passes=False),
    )


def kernel(x, table):
    xt = jnp.transpose(x)
    tt = jnp.transpose(table)
    tail = table[_NFULL * _TCOLS:].reshape(_TAIL // 4, 128)
    tw = _build_tr()(tt, tail)
    o = _build_emb()(xt, tw)
    return jnp.transpose(o, (2, 0, 1))
